# Initial kernel scaffold; baseline (speedup 1.0000x reference)
import jax
import jax.numpy as jnp
from jax.experimental import pallas as pl


def kernel(attention, W, b):
    raise NotImplementedError("write your pallas kernel here")


# speedup vs baseline: 56.8837x; 56.8837x over previous
import functools

import jax
import jax.numpy as jnp
from jax.experimental import pallas as pl

_B, _H, _S, _K, _L = 2, 12, 2048, 10, 45
_R = 16
_G = _S // _R
_C = 512


def _oddeven_merge(lo, n, r):
    step = r * 2
    if step < n:
        yield from _oddeven_merge(lo, n, step)
        yield from _oddeven_merge(lo + r, n, step)
        for i in range(lo + r, lo + n - r, step):
            yield (i, i + r)
    else:
        yield (lo, lo + r)


def _oddeven_merge_sort(lo, n):
    if n > 1:
        m = n // 2
        yield from _oddeven_merge_sort(lo, m)
        yield from _oddeven_merge_sort(lo + m, m)
        yield from _oddeven_merge(lo, n, 1)


_SORT16_PAIRS = tuple(_oddeven_merge_sort(0, _R))


def _topk_body(x_ref, w_ref, b_ref, o_ref):
    h = pl.program_id(2)
    x = x_ref[0, 0]

    regs = [x[i * _G:(i + 1) * _G, :] for i in range(_R)]

    for i, j in _SORT16_PAIRS:
        a, b = regs[i], regs[j]
        regs[i] = jnp.maximum(a, b)
        regs[j] = jnp.minimum(a, b)

    g = _G
    while g > 1:
        half = g // 2
        lo = [r[:half] for r in regs]
        hi = [r[half:] for r in regs]
        regs = [jnp.maximum(lo[i], hi[_R - 1 - i]) for i in range(_R)]
        for d in (8, 4, 2, 1):
            for i in range(_R):
                if (i % (2 * d)) < d:
                    a, b = regs[i], regs[i + d]
                    regs[i] = jnp.maximum(a, b)
                    regs[i + d] = jnp.minimum(a, b)
        g = half

    feat = jnp.concatenate(regs[:_K], axis=0)
    w = w_ref[0]
    contrib = jax.lax.dot_general(
        feat, w, (((0,), (1,)), ((), ())),
        preferred_element_type=jnp.float32,
    )

    @pl.when(h == 0)
    def _init():
        o_ref[0] = contrib + b_ref[...]

    @pl.when(h != 0)
    def _acc():
        o_ref[0] += contrib


@jax.jit
def kernel(attention, W, b):
    Wh = W.reshape(_L, _H, _K).transpose(1, 0, 2)
    b2 = b.reshape(1, _L)
    grid = (_B, _S // _C, _H)
    return pl.pallas_call(
        _topk_body,
        grid=grid,
        in_specs=[
            pl.BlockSpec((1, 1, _S, _C), lambda bb, cb, h: (bb, h, 0, cb)),
            pl.BlockSpec((1, _L, _K), lambda bb, cb, h: (h, 0, 0)),
            pl.BlockSpec((1, _L), lambda bb, cb, h: (0, 0)),
        ],
        out_specs=pl.BlockSpec((1, _C, _L), lambda bb, cb, h: (bb, cb, 0)),
        out_shape=jax.ShapeDtypeStruct((_B, _S, _L), jnp.float32),
    )(attention, Wh, b2)


# speedup vs baseline: 63.5929x; 1.1179x over previous
import functools

import jax
import jax.numpy as jnp
from jax import lax
from jax.experimental import pallas as pl
from jax.experimental.pallas import tpu as pltpu
from jax.experimental.pallas import tpu_sc as plsc

_B, _H, _S, _K, _L = 2, 12, 2048, 10, 45
_R = 16
_C = 512
_HT = 8
_MSC = _H - _HT
_NSC = _B * _MSC
_NW = 32
_CSC = 128
_RC = 512
_SUBS = _CSC // 16
_SPW = _NSC * (_S // _CSC) // _NW
_C2 = 512


def _oddeven_merge(lo, n, r):
    step = r * 2
    if step < n:
        yield from _oddeven_merge(lo, n, step)
        yield from _oddeven_merge(lo + r, n, step)
        for i in range(lo + r, lo + n - r, step):
            yield (i, i + r)
    else:
        yield (lo, lo + r)


def _oddeven_merge_sort(lo, n):
    if n > 1:
        m = n // 2
        yield from _oddeven_merge_sort(lo, m)
        yield from _oddeven_merge_sort(lo + m, m)
        yield from _oddeven_merge(lo, n, 1)


_SORT16_PAIRS = tuple(_oddeven_merge_sort(0, _R))


def _sort16(regs):
    for i, j in _SORT16_PAIRS:
        a, b = regs[i], regs[j]
        regs[i] = jnp.maximum(a, b)
        regs[j] = jnp.minimum(a, b)
    return regs


def _bitonic_clean16(regs):
    for d in (8, 4, 2, 1):
        for i in range(_R):
            if (i % (2 * d)) < d:
                a, b = regs[i], regs[i + d]
                regs[i] = jnp.maximum(a, b)
                regs[i + d] = jnp.minimum(a, b)
    return regs


def _tc_body(x_ref, w_ref, b_ref, o_ref):
    h = pl.program_id(2)
    x = x_ref[0, 0]
    g = _S // _R
    regs = _sort16([x[i * g:(i + 1) * g, :] for i in range(_R)])
    while g > 1:
        half = g // 2
        lo = [r[:half] for r in regs]
        hi = [r[half:] for r in regs]
        regs = _bitonic_clean16(
            [jnp.maximum(lo[i], hi[_R - 1 - i]) for i in range(_R)])
        g = half
    feat = jnp.concatenate(regs[:_K], axis=0)
    contrib = lax.dot_general(
        feat, w_ref[0], (((0,), (1,)), ((), ())),
        preferred_element_type=jnp.float32,
    )

    @pl.when(h == 0)
    def _init():
        o_ref[0] = contrib + b_ref[...]

    @pl.when(h != 0)
    def _acc():
        o_ref[0] += contrib


def _tc_partial(attention, Wh, b2):
    return pl.pallas_call(
        _tc_body,
        grid=(_B, _S // _C, _HT),
        in_specs=[
            pl.BlockSpec((1, 1, _S, _C), lambda bb, cb, h: (bb, h, 0, cb)),
            pl.BlockSpec((1, _L, _K), lambda bb, cb, h: (h, 0, 0)),
            pl.BlockSpec((1, _L), lambda bb, cb, h: (0, 0)),
        ],
        out_specs=pl.BlockSpec((1, _C, _L), lambda bb, cb, h: (bb, cb, 0)),
        out_shape=jax.ShapeDtypeStruct((_B, _S, _L), jnp.float32),
    )(attention, Wh, b2)


_SC_MESH = plsc.VectorSubcoreMesh(core_axis_name="c", subcore_axis_name="s")


@functools.partial(
    pl.kernel,
    out_type=jax.ShapeDtypeStruct((_NSC, _K, _S), jnp.float32),
    mesh=_SC_MESH,
    scratch_types=[
        pltpu.VMEM((_RC, _CSC), jnp.float32),
        pltpu.VMEM((_K, _CSC), jnp.float32),
    ],
)
def _sc_topk(att_hbm, feat_hbm, buf, fbuf):
    wid = lax.axis_index("s") * 2 + lax.axis_index("c")

    def strip_body(t, carry):
        sidx = wid * _SPW + t
        pair = sidx // (_S // _CSC)
        c0 = (sidx % (_S // _CSC)) * _CSC
        bh = (pair // _MSC) * _H + _HT + lax.rem(pair, _MSC)

        def chunk_body(ci, runs):
            pltpu.sync_copy(
                att_hbm.at[bh, pl.ds(ci * _RC, _RC), pl.ds(c0, _CSC)], buf)
            out = []
            for sub in range(_SUBS):
                def group_body(g, run, sub=sub):
                    regs = _sort16([
                        buf[_R * g + i, 16 * sub:16 * (sub + 1)]
                        for i in range(_R)
                    ])
                    merged = [
                        jnp.maximum(run[i], regs[_R - 1 - i])
                        for i in range(_R)
                    ]
                    return tuple(_bitonic_clean16(merged))

                out += list(lax.fori_loop(
                    0, _RC // _R, group_body,
                    tuple(runs[sub * _R:(sub + 1) * _R])))
            return tuple(out)

        init = tuple(
            jnp.full((16,), -jnp.inf, jnp.float32)
            for _ in range(_R * _SUBS))
        runs = lax.fori_loop(0, _S // _RC, chunk_body, init)
        for sub in range(_SUBS):
            for j in range(_K):
                fbuf[j, 16 * sub:16 * (sub + 1)] = runs[sub * _R + j]
        pltpu.sync_copy(fbuf, feat_hbm.at[pair, :, pl.ds(c0, _CSC)])
        return carry

    lax.fori_loop(0, _SPW, strip_body, 0)


def _combine_body(p_ref, f_ref, w_ref, o_ref):
    m = pl.program_id(2)
    contrib = lax.dot_general(
        f_ref[0, 0], w_ref[0], (((0,), (1,)), ((), ())),
        preferred_element_type=jnp.float32,
    )

    @pl.when(m == 0)
    def _init():
        o_ref[0] = p_ref[0] + contrib

    @pl.when(m != 0)
    def _acc():
        o_ref[0] += contrib


def _combine(partial, feats, Wsc):
    return pl.pallas_call(
        _combine_body,
        grid=(_B, _S // _C2, _MSC),
        in_specs=[
            pl.BlockSpec((1, _C2, _L), lambda bb, cb, m: (bb, cb, 0)),
            pl.BlockSpec((1, 1, _K, _C2), lambda bb, cb, m: (bb, m, 0, cb)),
            pl.BlockSpec((1, _L, _K), lambda bb, cb, m: (m, 0, 0)),
        ],
        out_specs=pl.BlockSpec((1, _C2, _L), lambda bb, cb, m: (bb, cb, 0)),
        out_shape=jax.ShapeDtypeStruct((_B, _S, _L), jnp.float32),
    )(partial, feats, Wsc)


@jax.jit
def kernel(attention, W, b):
    Wh = W.reshape(_L, _H, _K).transpose(1, 0, 2)
    b2 = b.reshape(1, _L)
    att3 = attention.reshape(_B * _H, _S, _S)
    feats = _sc_topk(att3)
    partial = _tc_partial(attention, Wh[:_HT], b2)
    return _combine(partial, feats.reshape(_B, _MSC, _K, _S), Wh[_HT:])


# speedup vs baseline: 63.6387x; 1.0007x over previous
import functools

import jax
import jax.numpy as jnp
from jax import lax
from jax.experimental import pallas as pl
from jax.experimental.pallas import tpu as pltpu
from jax.experimental.pallas import tpu_sc as plsc

_B, _H, _S, _K, _L = 2, 12, 2048, 10, 45
_R = 16
_C = 512
_HT = 8
_MSC = _H - _HT
_NSC = _B * _MSC
_NW = 32
_CSC = 128
_RC = 512
_SUBS = _CSC // 16
_SPW = _NSC * (_S // _CSC) // _NW
_C2 = 512


def _oddeven_merge(lo, n, r):
    step = r * 2
    if step < n:
        yield from _oddeven_merge(lo, n, step)
        yield from _oddeven_merge(lo + r, n, step)
        for i in range(lo + r, lo + n - r, step):
            yield (i, i + r)
    else:
        yield (lo, lo + r)


def _oddeven_merge_sort(lo, n):
    if n > 1:
        m = n // 2
        yield from _oddeven_merge_sort(lo, m)
        yield from _oddeven_merge_sort(lo + m, m)
        yield from _oddeven_merge(lo, n, 1)


_SORT16_PAIRS = tuple(_oddeven_merge_sort(0, _R))


def _sort16(regs):
    for i, j in _SORT16_PAIRS:
        a, b = regs[i], regs[j]
        regs[i] = jnp.maximum(a, b)
        regs[j] = jnp.minimum(a, b)
    return regs


def _bitonic_clean16(regs):
    for d in (8, 4, 2, 1):
        for i in range(_R):
            if (i % (2 * d)) < d:
                a, b = regs[i], regs[i + d]
                regs[i] = jnp.maximum(a, b)
                regs[i + d] = jnp.minimum(a, b)
    return regs


def _tc_body(x_ref, w_ref, b_ref, o_ref):
    h = pl.program_id(2)
    x = x_ref[0, 0]
    cols = []
    for c0 in range(0, _C, 128):
        run = None
        for t in range(_S // 128):
            base = t * 128
            regs = _sort16([
                x[base + 8 * i:base + 8 * (i + 1), c0:c0 + 128]
                for i in range(_R)
            ])
            if run is None:
                run = regs
            else:
                run = _bitonic_clean16([
                    jnp.maximum(run[i], regs[_R - 1 - i]) for i in range(_R)
                ])
        g = 8
        while g > 1:
            half = g // 2
            lo = [r[:half] for r in run]
            hi = [r[half:] for r in run]
            run = _bitonic_clean16(
                [jnp.maximum(lo[i], hi[_R - 1 - i]) for i in range(_R)])
            g = half
        cols.append(jnp.concatenate(run[:_K], axis=0))
    feat = jnp.concatenate(cols, axis=1)
    contrib = lax.dot_general(
        feat, w_ref[0], (((0,), (1,)), ((), ())),
        preferred_element_type=jnp.float32,
    )

    @pl.when(h == 0)
    def _init():
        o_ref[0] = contrib + b_ref[...]

    @pl.when(h != 0)
    def _acc():
        o_ref[0] += contrib


def _tc_partial(attention, Wh, b2):
    return pl.pallas_call(
        _tc_body,
        grid=(_B, _S // _C, _HT),
        in_specs=[
            pl.BlockSpec((1, 1, _S, _C), lambda bb, cb, h: (bb, h, 0, cb)),
            pl.BlockSpec((1, _L, _K), lambda bb, cb, h: (h, 0, 0)),
            pl.BlockSpec((1, _L), lambda bb, cb, h: (0, 0)),
        ],
        out_specs=pl.BlockSpec((1, _C, _L), lambda bb, cb, h: (bb, cb, 0)),
        out_shape=jax.ShapeDtypeStruct((_B, _S, _L), jnp.float32),
    )(attention, Wh, b2)


_SC_MESH = plsc.VectorSubcoreMesh(core_axis_name="c", subcore_axis_name="s")


@functools.partial(
    pl.kernel,
    out_type=jax.ShapeDtypeStruct((_NSC, _K, _S), jnp.float32),
    mesh=_SC_MESH,
    scratch_types=[
        pltpu.VMEM((_RC, _CSC), jnp.float32),
        pltpu.VMEM((_K, _CSC), jnp.float32),
    ],
)
def _sc_topk(att_hbm, feat_hbm, buf, fbuf):
    wid = lax.axis_index("s") * 2 + lax.axis_index("c")

    def strip_body(t, carry):
        sidx = wid * _SPW + t
        pair = sidx // (_S // _CSC)
        c0 = (sidx % (_S // _CSC)) * _CSC
        bh = (pair // _MSC) * _H + _HT + lax.rem(pair, _MSC)

        def chunk_body(ci, runs):
            pltpu.sync_copy(
                att_hbm.at[bh, pl.ds(ci * _RC, _RC), pl.ds(c0, _CSC)], buf)
            out = []
            for sub in range(_SUBS):
                def group_body(g, run, sub=sub):
                    regs = _sort16([
                        buf[_R * g + i, 16 * sub:16 * (sub + 1)]
                        for i in range(_R)
                    ])
                    merged = [
                        jnp.maximum(run[i], regs[_R - 1 - i])
                        for i in range(_R)
                    ]
                    return tuple(_bitonic_clean16(merged))

                out += list(lax.fori_loop(
                    0, _RC // _R, group_body,
                    tuple(runs[sub * _R:(sub + 1) * _R])))
            return tuple(out)

        init = tuple(
            jnp.full((16,), -jnp.inf, jnp.float32)
            for _ in range(_R * _SUBS))
        runs = lax.fori_loop(0, _S // _RC, chunk_body, init)
        for sub in range(_SUBS):
            for j in range(_K):
                fbuf[j, 16 * sub:16 * (sub + 1)] = runs[sub * _R + j]
        pltpu.sync_copy(fbuf, feat_hbm.at[pair, :, pl.ds(c0, _CSC)])
        return carry

    lax.fori_loop(0, _SPW, strip_body, 0)


def _combine_body(p_ref, f_ref, w_ref, o_ref):
    m = pl.program_id(2)
    contrib = lax.dot_general(
        f_ref[0, 0], w_ref[0], (((0,), (1,)), ((), ())),
        preferred_element_type=jnp.float32,
    )

    @pl.when(m == 0)
    def _init():
        o_ref[0] = p_ref[0] + contrib

    @pl.when(m != 0)
    def _acc():
        o_ref[0] += contrib


def _combine(partial, feats, Wsc):
    return pl.pallas_call(
        _combine_body,
        grid=(_B, _S // _C2, _MSC),
        in_specs=[
            pl.BlockSpec((1, _C2, _L), lambda bb, cb, m: (bb, cb, 0)),
            pl.BlockSpec((1, 1, _K, _C2), lambda bb, cb, m: (bb, m, 0, cb)),
            pl.BlockSpec((1, _L, _K), lambda bb, cb, m: (m, 0, 0)),
        ],
        out_specs=pl.BlockSpec((1, _C2, _L), lambda bb, cb, m: (bb, cb, 0)),
        out_shape=jax.ShapeDtypeStruct((_B, _S, _L), jnp.float32),
    )(partial, feats, Wsc)


@jax.jit
def kernel(attention, W, b):
    Wh = W.reshape(_L, _H, _K).transpose(1, 0, 2)
    b2 = b.reshape(1, _L)
    att3 = attention.reshape(_B * _H, _S, _S)
    feats = _sc_topk(att3)
    partial = _tc_partial(attention, Wh[:_HT], b2)
    return _combine(partial, feats.reshape(_B, _MSC, _K, _S), Wh[_HT:])


# speedup vs baseline: 79.8822x; 1.2552x over previous
import functools

import jax
import jax.numpy as jnp
from jax import lax
from jax.experimental import pallas as pl
from jax.experimental.pallas import tpu as pltpu
from jax.experimental.pallas import tpu_sc as plsc

_B, _H, _S, _K, _L = 2, 12, 2048, 10, 45
_R = 16
_C = 512
_HT = 9
_MSC = _H - _HT
_NSC = _B * _MSC
_NW = 32
_CSC = 128
_RC = 512
_SUBS = _CSC // 16
_SPW = _NSC * (_S // _CSC) // _NW
_C2 = 512


def _oddeven_merge(lo, n, r):
    step = r * 2
    if step < n:
        yield from _oddeven_merge(lo, n, step)
        yield from _oddeven_merge(lo + r, n, step)
        for i in range(lo + r, lo + n - r, step):
            yield (i, i + r)
    else:
        yield (lo, lo + r)


def _oddeven_merge_sort(lo, n):
    if n > 1:
        m = n // 2
        yield from _oddeven_merge_sort(lo, m)
        yield from _oddeven_merge_sort(lo + m, m)
        yield from _oddeven_merge(lo, n, 1)


_SORT16_PAIRS = tuple(_oddeven_merge_sort(0, _R))


def _sort16(regs):
    for i, j in _SORT16_PAIRS:
        a, b = regs[i], regs[j]
        regs[i] = jnp.maximum(a, b)
        regs[j] = jnp.minimum(a, b)
    return regs


def _bitonic_clean16(regs):
    for d in (8, 4, 2, 1):
        for i in range(_R):
            if (i % (2 * d)) < d:
                a, b = regs[i], regs[i + d]
                regs[i] = jnp.maximum(a, b)
                regs[i + d] = jnp.minimum(a, b)
    return regs


def _tc_body(x_ref, w_ref, b_ref, o_ref):
    h = pl.program_id(2)
    x = x_ref[0, 0]
    cols = []
    for c0 in range(0, _C, 128):
        run = None
        for t in range(_S // 128):
            base = t * 128
            regs = _sort16([
                x[base + 8 * i:base + 8 * (i + 1), c0:c0 + 128]
                for i in range(_R)
            ])
            if run is None:
                run = regs
            else:
                run = _bitonic_clean16([
                    jnp.maximum(run[i], regs[_R - 1 - i]) for i in range(_R)
                ])
        g = 8
        while g > 1:
            half = g // 2
            lo = [r[:half] for r in run]
            hi = [r[half:] for r in run]
            run = _bitonic_clean16(
                [jnp.maximum(lo[i], hi[_R - 1 - i]) for i in range(_R)])
            g = half
        cols.append(jnp.concatenate(run[:_K], axis=0))
    feat = jnp.concatenate(cols, axis=1)
    contrib = lax.dot_general(
        feat, w_ref[0], (((0,), (1,)), ((), ())),
        preferred_element_type=jnp.float32,
    )

    @pl.when(h == 0)
    def _init():
        o_ref[0] = contrib + b_ref[...]

    @pl.when(h != 0)
    def _acc():
        o_ref[0] += contrib


def _tc_partial(attention, Wh, b2):
    return pl.pallas_call(
        _tc_body,
        grid=(_B, _S // _C, _HT),
        in_specs=[
            pl.BlockSpec((1, 1, _S, _C), lambda bb, cb, h: (bb, h, 0, cb)),
            pl.BlockSpec((1, _L, _K), lambda bb, cb, h: (h, 0, 0)),
            pl.BlockSpec((1, _L), lambda bb, cb, h: (0, 0)),
        ],
        out_specs=pl.BlockSpec((1, _C, _L), lambda bb, cb, h: (bb, cb, 0)),
        out_shape=jax.ShapeDtypeStruct((_B, _S, _L), jnp.float32),
    )(attention, Wh, b2)


_SC_MESH = plsc.VectorSubcoreMesh(core_axis_name="c", subcore_axis_name="s")


@functools.partial(
    pl.kernel,
    out_type=jax.ShapeDtypeStruct((_NSC, _K, _S), jnp.float32),
    mesh=_SC_MESH,
    scratch_types=[
        pltpu.VMEM((_RC, _CSC), jnp.float32),
        pltpu.VMEM((_K, _CSC), jnp.float32),
    ],
)
def _sc_topk(att_hbm, feat_hbm, buf, fbuf):
    wid = lax.axis_index("s") * 2 + lax.axis_index("c")

    def strip_body(t, carry):
        sidx = wid * _SPW + t
        pair = sidx // (_S // _CSC)
        c0 = (sidx % (_S // _CSC)) * _CSC
        bh = (pair // _MSC) * _H + _HT + lax.rem(pair, _MSC)

        def chunk_body(ci, runs):
            pltpu.sync_copy(
                att_hbm.at[bh, pl.ds(ci * _RC, _RC), pl.ds(c0, _CSC)], buf)
            out = []
            for sub in range(_SUBS):
                def group_body(g, run, sub=sub):
                    regs = _sort16([
                        buf[_R * g + i, 16 * sub:16 * (sub + 1)]
                        for i in range(_R)
                    ])
                    merged = [
                        jnp.maximum(run[i], regs[_R - 1 - i])
                        for i in range(_R)
                    ]
                    return tuple(_bitonic_clean16(merged))

                out += list(lax.fori_loop(
                    0, _RC // _R, group_body,
                    tuple(runs[sub * _R:(sub + 1) * _R])))
            return tuple(out)

        init = tuple(
            jnp.full((16,), -jnp.inf, jnp.float32)
            for _ in range(_R * _SUBS))
        runs = lax.fori_loop(0, _S // _RC, chunk_body, init)
        for sub in range(_SUBS):
            for j in range(_K):
                fbuf[j, 16 * sub:16 * (sub + 1)] = runs[sub * _R + j]
        pltpu.sync_copy(fbuf, feat_hbm.at[pair, :, pl.ds(c0, _CSC)])
        return carry

    lax.fori_loop(0, _SPW, strip_body, 0)


def _combine_body(p_ref, f_ref, w_ref, o_ref):
    m = pl.program_id(2)
    contrib = lax.dot_general(
        f_ref[0, 0], w_ref[0], (((0,), (1,)), ((), ())),
        preferred_element_type=jnp.float32,
    )

    @pl.when(m == 0)
    def _init():
        o_ref[0] = p_ref[0] + contrib

    @pl.when(m != 0)
    def _acc():
        o_ref[0] += contrib


def _combine(partial, feats, Wsc):
    return pl.pallas_call(
        _combine_body,
        grid=(_B, _S // _C2, _MSC),
        in_specs=[
            pl.BlockSpec((1, _C2, _L), lambda bb, cb, m: (bb, cb, 0)),
            pl.BlockSpec((1, 1, _K, _C2), lambda bb, cb, m: (bb, m, 0, cb)),
            pl.BlockSpec((1, _L, _K), lambda bb, cb, m: (m, 0, 0)),
        ],
        out_specs=pl.BlockSpec((1, _C2, _L), lambda bb, cb, m: (bb, cb, 0)),
        out_shape=jax.ShapeDtypeStruct((_B, _S, _L), jnp.float32),
    )(partial, feats, Wsc)


@jax.jit
def kernel(attention, W, b):
    Wh = W.reshape(_L, _H, _K).transpose(1, 0, 2)
    b2 = b.reshape(1, _L)
    att3 = attention.reshape(_B * _H, _S, _S)
    feats = _sc_topk(att3)
    partial = _tc_partial(attention, Wh[:_HT], b2)
    return _combine(partial, feats.reshape(_B, _MSC, _K, _S), Wh[_HT:])
